# baseline (speedup 1.0000x reference)
import functools

import jax
import jax.numpy as jnp
from jax import lax
from jax.experimental import pallas as pl
from jax.experimental.pallas import tpu as pltpu
from jax.experimental.pallas import tpu_sc as plsc

VOCAB_N = 1000
VPAD = 1024
ROWS_N = 51200
NC = 2
NS = 16
NW = NC * NS
PER_W = ROWS_N // NW
LCH = 80
NVG = VOCAB_N // 8
NU = ROWS_N // 128
IDX_PAD = 51456


def _lse_body(tpad_ref, out_ref):
    x = tpad_ref[...]
    m = jnp.max(x, axis=1)
    s = jnp.sum(jnp.exp(x - m[:, None]), axis=1)
    out_ref[...] = m + jnp.log(s)


def _mean_body(part_ref, out_ref):
    out_ref[...] = (jnp.sum(part_ref[...]) * (1.0 / ROWS_N))[None, None]


def _sc_body(idx_hbm, tgt_hbm, lse_hbm, tflat_hbm, gflat_hbm,
             out_hbm, part_hbm,
             idxg_v, idx_v, tgt_v, fidx_v, lsei_v, tlog_v,
             g0, g1, st0, st1, st2, st3, part_v,
             semg0, semg1, semw0, semw1, semw2, semw3, seml):
    wid = lax.axis_index("s") * NC + lax.axis_index("c")

    base = wid * PER_W
    pltpu.sync_copy(idx_hbm.at[pl.ds(base, PER_W)], idx_v)
    pltpu.sync_copy(tgt_hbm.at[pl.ds(base, PER_W)], tgt_v)

    def fidx_body(j, _):
        s = pl.ds(j * 16, 16)
        fidx_v[s] = idx_v[s] * VPAD + tgt_v[s]
        return 0

    lax.fori_loop(0, PER_W // 16, fidx_body, 0)

    def lgather_body(j, _):
        s = pl.ds(j * LCH, LCH)
        pltpu.async_copy(lse_hbm.at[idx_v.at[s]], lsei_v.at[s], seml).wait()
        pltpu.async_copy(tflat_hbm.at[fidx_v.at[s]], tlog_v.at[s],
                         seml).wait()
        return 0

    lax.fori_loop(0, PER_W // LCH, lgather_body, 0)

    def loss_body(j, acc):
        s = pl.ds(j * 16, 16)
        return acc + (lsei_v[s] - tlog_v[s])

    acc = lax.fori_loop(0, PER_W // 16, loss_body,
                        jnp.zeros((16,), jnp.float32))
    part_v[...] = acc
    pltpu.sync_copy(part_v, part_hbm.at[wid])

    nu = jnp.where(wid < 16, 13, 12)
    u0 = jnp.where(wid < 16, wid * 13, wid * 12 + 16)
    i0 = u0 * 128
    pltpu.sync_copy(idx_hbm.at[pl.ds(i0, 1664)], idxg_v)

    gbuf = (g0, g1)
    semg = (semg0, semg1)
    stage = (st0, st1, st2, st3)
    semw = (semw0, semw1, semw2, semw3)

    def g_load(vg, k):
        return pltpu.async_copy(
            gflat_hbm.at[pl.ds(vg * (8 * VPAD), 8 * VPAD)], gbuf[k], semg[k])

    g_load(0, 0)

    def build_tile(gk, vg, t, s):
        gcount = vg * nu + t

        @pl.when(gcount >= 4)
        def _():
            pltpu.make_async_copy(
                stage[s],
                out_hbm.at[pl.ds(pl.multiple_of(vg * 8, 8), 8),
                           pl.ds(pl.multiple_of(i0 + t * 128, 128), 128)],
                semw[s]).wait()

        for vr in range(8):
            for grp in range(8):
                fidx = idxg_v[pl.ds(t * 128 + grp * 16, 16)] + vr * VPAD
                stage[s][vr, pl.ds(grp * 16, 16)] = plsc.load_gather(
                    gbuf[gk], [fidx])
        pltpu.async_copy(
            stage[s],
            out_hbm.at[pl.ds(pl.multiple_of(vg * 8, 8), 8),
                       pl.ds(pl.multiple_of(i0 + t * 128, 128), 128)],
            semw[s])

    def do_vgroup(gk, vg):
        @pl.when(vg + 1 < NVG)
        def _():
            g_load(vg + 1, 1 - gk)

        pltpu.make_async_copy(
            gflat_hbm.at[pl.ds(vg * (8 * VPAD), 8 * VPAD)],
            gbuf[gk], semg[gk]).wait()

        def unit4(j, _):
            for u in range(4):
                build_tile(gk, vg, j * 4 + u, u)
            return 0

        lax.fori_loop(0, 3, unit4, 0)

        @pl.when(nu == 13)
        def _():
            build_tile(gk, vg, 12, 0)

    def vg_pair(o, _):
        @pl.when(o * 2 < NVG)
        def _():
            do_vgroup(0, o * 2)

        @pl.when(o * 2 + 1 < NVG)
        def _():
            do_vgroup(1, o * 2 + 1)

        return 0

    lax.fori_loop(0, (NVG + 1) // 2, vg_pair, 0)

    for s in range(4):
        pltpu.make_async_copy(
            stage[s],
            out_hbm.at[pl.ds(pl.multiple_of(0 * 8, 8), 8),
                       pl.ds(pl.multiple_of(i0, 128), 128)],
            semw[s]).wait()


_sc_gather = functools.partial(
    pl.kernel,
    out_type=[
        jax.ShapeDtypeStruct((VOCAB_N, ROWS_N), jnp.float32),
        jax.ShapeDtypeStruct((NW, 16), jnp.float32),
    ],
    mesh=plsc.VectorSubcoreMesh(core_axis_name="c", subcore_axis_name="s"),
    compiler_params=pltpu.CompilerParams(use_tc_tiling_on_sc=True,
                                         needs_layout_passes=False,
                                         disable_bounds_checks=True),
    scratch_types=[
        pltpu.VMEM((1664,), jnp.int32),
        pltpu.VMEM((PER_W,), jnp.int32),
        pltpu.VMEM((PER_W,), jnp.int32),
        pltpu.VMEM((PER_W,), jnp.int32),
        pltpu.VMEM((PER_W,), jnp.float32),
        pltpu.VMEM((PER_W,), jnp.float32),
        pltpu.VMEM((8 * VPAD,), jnp.float32),
        pltpu.VMEM((8 * VPAD,), jnp.float32),
        pltpu.VMEM((8, 128), jnp.float32),
        pltpu.VMEM((8, 128), jnp.float32),
        pltpu.VMEM((8, 128), jnp.float32),
        pltpu.VMEM((8, 128), jnp.float32),
        pltpu.VMEM((16,), jnp.float32),
        pltpu.SemaphoreType.DMA,
        pltpu.SemaphoreType.DMA,
        pltpu.SemaphoreType.DMA,
        pltpu.SemaphoreType.DMA,
        pltpu.SemaphoreType.DMA,
        pltpu.SemaphoreType.DMA,
        pltpu.SemaphoreType.DMA,
    ],
)(_sc_body)


def kernel(idx, targets, table):
    idx_f = jnp.pad(idx.reshape(-1).astype(jnp.int32), (0, IDX_PAD - ROWS_N))
    tgt_f = jnp.pad(targets.reshape(-1).astype(jnp.int32),
                    (0, IDX_PAD - ROWS_N))
    cpad = VPAD - VOCAB_N
    tpad_sq = jnp.pad(table, ((0, cpad), (0, cpad)), constant_values=-1e30)
    tflat = jnp.pad(table, ((0, 0), (0, cpad))).reshape(-1)
    gflat = jnp.pad(table.T, ((0, 0), (0, cpad))).reshape(-1)
    lse = pl.pallas_call(
        _lse_body,
        out_shape=jax.ShapeDtypeStruct((VPAD,), jnp.float32),
    )(tpad_sq)
    logits_t, part = _sc_gather(idx_f, tgt_f, lse, tflat, gflat)
    loss = pl.pallas_call(
        _mean_body,
        out_shape=jax.ShapeDtypeStruct((1, 1), jnp.float32),
    )(part)
    return logits_t.T, loss[0, 0]


# speedup vs baseline: 2.8105x; 2.8105x over previous
import functools

import jax
import jax.numpy as jnp
from jax import lax
from jax.experimental import pallas as pl
from jax.experimental.pallas import tpu as pltpu
from jax.experimental.pallas import tpu_sc as plsc

VOCAB_N = 1000
VPAD = 1024
ROWS_N = 51200
NC = 2
NS = 16
NW = NC * NS
PER_W = ROWS_N // NW
LCH = 80
NVG = VOCAB_N // 8
NU = ROWS_N // 128
IDX_PAD = 51456


def _lse_body(tpad_ref, out_ref):
    x = tpad_ref[...]
    m = jnp.max(x, axis=1)
    s = jnp.sum(jnp.exp(x - m[:, None]), axis=1)
    out_ref[...] = m + jnp.log(s)


def _mean_body(part_ref, out_ref):
    out_ref[...] = (jnp.sum(part_ref[...]) * (1.0 / ROWS_N))[None, None]


def _sc_body(idx_hbm, tgt_hbm, lse_hbm, tflat_hbm, gflat_hbm,
             out_hbm, part_hbm,
             idxg_v, idx_v, tgt_v, fidx_v, lsei_v, tlog_v,
             g0, g1, st0, st1, st2, st3, part_v,
             semg0, semg1, semw0, semw1, semw2, semw3, seml):
    wid = lax.axis_index("s") * NC + lax.axis_index("c")

    base = wid * PER_W
    pltpu.sync_copy(idx_hbm.at[pl.ds(base, PER_W)], idx_v)
    pltpu.sync_copy(tgt_hbm.at[pl.ds(base, PER_W)], tgt_v)

    def fidx_body(j, _):
        s = pl.ds(j * 16, 16)
        fidx_v[s] = idx_v[s] * VPAD + tgt_v[s]
        return 0

    lax.fori_loop(0, PER_W // 16, fidx_body, 0)

    def lgather_body(j, _):
        s = pl.ds(j * LCH, LCH)
        pltpu.async_copy(lse_hbm.at[idx_v.at[s]], lsei_v.at[s], seml).wait()
        pltpu.async_copy(tflat_hbm.at[fidx_v.at[s]], tlog_v.at[s],
                         seml).wait()
        return 0

    lax.fori_loop(0, PER_W // LCH, lgather_body, 0)

    def loss_body(j, acc):
        s = pl.ds(j * 16, 16)
        return acc + (lsei_v[s] - tlog_v[s])

    acc = lax.fori_loop(0, PER_W // 16, loss_body,
                        jnp.zeros((16,), jnp.float32))
    part_v[...] = acc
    pltpu.sync_copy(part_v, part_hbm.at[wid])

    nu = jnp.where(wid < 16, 13, 12)
    u0 = jnp.where(wid < 16, wid * 13, wid * 12 + 16)
    i0 = u0 * 128
    pltpu.sync_copy(idx_hbm.at[pl.ds(i0, 1664)], idxg_v)

    gbuf = (g0, g1)
    semg = (semg0, semg1)
    stage = (st0, st1, st2, st3)
    semw = (semw0, semw1, semw2, semw3)

    def g_load(vg, k):
        return pltpu.async_copy(
            gflat_hbm.at[pl.ds(vg * (8 * VPAD), 8 * VPAD)], gbuf[k], semg[k])

    g_load(0, 0)

    def build_tile(gk, vg, t, s):
        gcount = vg * nu + t

        @pl.when(gcount >= 4)
        def _():
            pltpu.make_async_copy(
                stage[s],
                out_hbm.at[pl.ds(pl.multiple_of(vg * 8, 8), 8),
                           pl.ds(pl.multiple_of(i0 + t * 128, 128), 128)],
                semw[s]).wait()

        for grp in range(8):
            iv = idxg_v[pl.ds(t * 128 + grp * 16, 16)]
            vals = [plsc.load_gather(gbuf[gk], [iv + vr * VPAD])
                    for vr in range(8)]
            for vr in range(8):
                stage[s][vr, pl.ds(grp * 16, 16)] = vals[vr]
        pltpu.async_copy(
            stage[s],
            out_hbm.at[pl.ds(pl.multiple_of(vg * 8, 8), 8),
                       pl.ds(pl.multiple_of(i0 + t * 128, 128), 128)],
            semw[s])

    def do_vgroup(gk, vg):
        @pl.when(vg + 1 < NVG)
        def _():
            g_load(vg + 1, 1 - gk)

        pltpu.make_async_copy(
            gflat_hbm.at[pl.ds(vg * (8 * VPAD), 8 * VPAD)],
            gbuf[gk], semg[gk]).wait()

        def unit4(j, _):
            for u in range(4):
                build_tile(gk, vg, j * 4 + u, u)
            return 0

        lax.fori_loop(0, 3, unit4, 0)

        @pl.when(nu == 13)
        def _():
            build_tile(gk, vg, 12, 0)

    def vg_pair(o, _):
        @pl.when(o * 2 < NVG)
        def _():
            do_vgroup(0, o * 2)

        @pl.when(o * 2 + 1 < NVG)
        def _():
            do_vgroup(1, o * 2 + 1)

        return 0

    lax.fori_loop(0, (NVG + 1) // 2, vg_pair, 0)

    for s in range(4):
        pltpu.make_async_copy(
            stage[s],
            out_hbm.at[pl.ds(pl.multiple_of(0 * 8, 8), 8),
                       pl.ds(pl.multiple_of(i0, 128), 128)],
            semw[s]).wait()


_sc_gather = functools.partial(
    pl.kernel,
    out_type=[
        jax.ShapeDtypeStruct((VOCAB_N, ROWS_N), jnp.float32),
        jax.ShapeDtypeStruct((NW, 16), jnp.float32),
    ],
    mesh=plsc.VectorSubcoreMesh(core_axis_name="c", subcore_axis_name="s"),
    compiler_params=pltpu.CompilerParams(use_tc_tiling_on_sc=True,
                                         needs_layout_passes=False,
                                         disable_bounds_checks=True),
    scratch_types=[
        pltpu.VMEM((1664,), jnp.int32),
        pltpu.VMEM((PER_W,), jnp.int32),
        pltpu.VMEM((PER_W,), jnp.int32),
        pltpu.VMEM((PER_W,), jnp.int32),
        pltpu.VMEM((PER_W,), jnp.float32),
        pltpu.VMEM((PER_W,), jnp.float32),
        pltpu.VMEM((8 * VPAD,), jnp.float32),
        pltpu.VMEM((8 * VPAD,), jnp.float32),
        pltpu.VMEM((8, 128), jnp.float32),
        pltpu.VMEM((8, 128), jnp.float32),
        pltpu.VMEM((8, 128), jnp.float32),
        pltpu.VMEM((8, 128), jnp.float32),
        pltpu.VMEM((16,), jnp.float32),
        pltpu.SemaphoreType.DMA,
        pltpu.SemaphoreType.DMA,
        pltpu.SemaphoreType.DMA,
        pltpu.SemaphoreType.DMA,
        pltpu.SemaphoreType.DMA,
        pltpu.SemaphoreType.DMA,
        pltpu.SemaphoreType.DMA,
    ],
)(_sc_body)


def kernel(idx, targets, table):
    idx_f = jnp.pad(idx.reshape(-1).astype(jnp.int32), (0, IDX_PAD - ROWS_N))
    tgt_f = jnp.pad(targets.reshape(-1).astype(jnp.int32),
                    (0, IDX_PAD - ROWS_N))
    cpad = VPAD - VOCAB_N
    tpad_sq = jnp.pad(table, ((0, cpad), (0, cpad)), constant_values=-1e30)
    tflat = jnp.pad(table, ((0, 0), (0, cpad))).reshape(-1)
    gflat = jnp.pad(table.T, ((0, 0), (0, cpad))).reshape(-1)
    lse = pl.pallas_call(
        _lse_body,
        out_shape=jax.ShapeDtypeStruct((VPAD,), jnp.float32),
    )(tpad_sq)
    logits_t, part = _sc_gather(idx_f, tgt_f, lse, tflat, gflat)
    loss = pl.pallas_call(
        _mean_body,
        out_shape=jax.ShapeDtypeStruct((1, 1), jnp.float32),
    )(part)
    return logits_t.T, loss[0, 0]


# speedup vs baseline: 4.0724x; 1.4490x over previous
import functools

import jax
import jax.numpy as jnp
from jax import lax
from jax.experimental import pallas as pl
from jax.experimental.pallas import tpu as pltpu
from jax.experimental.pallas import tpu_sc as plsc

VOCAB_N = 1000
VPAD = 1024
ROWS_N = 51200
NC = 2
NS = 16
NW = NC * NS
PER_W = ROWS_N // NW
LCH = 80
NVG = VOCAB_N // 8
NU = ROWS_N // 128
IDX_PAD = 51456


def _lse_body(tpad_ref, out_ref):
    x = tpad_ref[...]
    m = jnp.max(x, axis=1)
    s = jnp.sum(jnp.exp(x - m[:, None]), axis=1)
    out_ref[...] = m + jnp.log(s)


def _mean_body(part_ref, out_ref):
    out_ref[...] = (jnp.sum(part_ref[...]) * (1.0 / ROWS_N))[None, None]


def _sc_body(idx_hbm, tgt_hbm, lse_hbm, tflat_hbm, gflat_hbm,
             out_hbm, part_hbm,
             idxg_v, idx_v, tgt_v, fidx_v, lsei_v, tlog_v,
             g0, g1, st0, st1, st2, st3, part_v,
             semg0, semg1, semw0, semw1, semw2, semw3, seml):
    wid = lax.axis_index("s") * NC + lax.axis_index("c")

    base = wid * PER_W
    pltpu.sync_copy(idx_hbm.at[pl.ds(base, PER_W)], idx_v)
    pltpu.sync_copy(tgt_hbm.at[pl.ds(base, PER_W)], tgt_v)

    def fidx_body(j, _):
        s = pl.ds(j * 16, 16)
        fidx_v[s] = idx_v[s] * VPAD + tgt_v[s]
        return 0

    lax.fori_loop(0, PER_W // 16, fidx_body, 0)

    def lgather_body(j, _):
        s = pl.ds(j * LCH, LCH)
        pltpu.async_copy(lse_hbm.at[idx_v.at[s]], lsei_v.at[s], seml).wait()
        pltpu.async_copy(tflat_hbm.at[fidx_v.at[s]], tlog_v.at[s],
                         seml).wait()
        return 0

    lax.fori_loop(0, PER_W // LCH, lgather_body, 0)

    def loss_body(j, acc):
        s = pl.ds(j * 16, 16)
        return acc + (lsei_v[s] - tlog_v[s])

    acc = lax.fori_loop(0, PER_W // 16, loss_body,
                        jnp.zeros((16,), jnp.float32))
    part_v[...] = acc
    pltpu.sync_copy(part_v, part_hbm.at[wid])

    nu = jnp.where(wid < 16, 13, 12)
    u0 = jnp.where(wid < 16, wid * 13, wid * 12 + 16)
    i0 = u0 * 128
    pltpu.sync_copy(idx_hbm.at[pl.ds(i0, 1664)], idxg_v)

    gbuf = (g0, g1)
    semg = (semg0, semg1)
    stage = (st0, st1, st2, st3)
    semw = (semw0, semw1, semw2, semw3)

    def g_load(vg, k):
        return pltpu.async_copy(
            gflat_hbm.at[pl.ds(vg * (8 * VPAD), 8 * VPAD)], gbuf[k], semg[k])

    g_load(0, 0)

    def build_tile(gk, vg, t, s):
        gcount = vg * nu + t

        @pl.when(gcount >= 4)
        def _():
            pltpu.make_async_copy(
                stage[s],
                out_hbm.at[pl.ds(pl.multiple_of(vg * 8, 8), 8),
                           pl.ds(pl.multiple_of(i0 + t * 128, 128), 128)],
                semw[s]).wait()

        @plsc.parallel_loop(0, 8, step=1, unroll=8)
        def _(grp):
            iv = idxg_v[pl.ds(t * 128 + grp * 16, 16)]
            for vr in range(8):
                stage[s][vr, pl.ds(grp * 16, 16)] = plsc.load_gather(
                    gbuf[gk], [iv + vr * VPAD])
        pltpu.async_copy(
            stage[s],
            out_hbm.at[pl.ds(pl.multiple_of(vg * 8, 8), 8),
                       pl.ds(pl.multiple_of(i0 + t * 128, 128), 128)],
            semw[s])

    def do_vgroup(gk, vg):
        @pl.when(vg + 1 < NVG)
        def _():
            g_load(vg + 1, 1 - gk)

        pltpu.make_async_copy(
            gflat_hbm.at[pl.ds(vg * (8 * VPAD), 8 * VPAD)],
            gbuf[gk], semg[gk]).wait()

        def unit4(j, _):
            for u in range(4):
                build_tile(gk, vg, j * 4 + u, u)
            return 0

        lax.fori_loop(0, 3, unit4, 0)

        @pl.when(nu == 13)
        def _():
            build_tile(gk, vg, 12, 0)

    def vg_pair(o, _):
        @pl.when(o * 2 < NVG)
        def _():
            do_vgroup(0, o * 2)

        @pl.when(o * 2 + 1 < NVG)
        def _():
            do_vgroup(1, o * 2 + 1)

        return 0

    lax.fori_loop(0, (NVG + 1) // 2, vg_pair, 0)

    for s in range(4):
        pltpu.make_async_copy(
            stage[s],
            out_hbm.at[pl.ds(pl.multiple_of(0 * 8, 8), 8),
                       pl.ds(pl.multiple_of(i0, 128), 128)],
            semw[s]).wait()


_sc_gather = functools.partial(
    pl.kernel,
    out_type=[
        jax.ShapeDtypeStruct((VOCAB_N, ROWS_N), jnp.float32),
        jax.ShapeDtypeStruct((NW, 16), jnp.float32),
    ],
    mesh=plsc.VectorSubcoreMesh(core_axis_name="c", subcore_axis_name="s"),
    compiler_params=pltpu.CompilerParams(use_tc_tiling_on_sc=True,
                                         needs_layout_passes=False,
                                         disable_bounds_checks=True),
    scratch_types=[
        pltpu.VMEM((1664,), jnp.int32),
        pltpu.VMEM((PER_W,), jnp.int32),
        pltpu.VMEM((PER_W,), jnp.int32),
        pltpu.VMEM((PER_W,), jnp.int32),
        pltpu.VMEM((PER_W,), jnp.float32),
        pltpu.VMEM((PER_W,), jnp.float32),
        pltpu.VMEM((8 * VPAD,), jnp.float32),
        pltpu.VMEM((8 * VPAD,), jnp.float32),
        pltpu.VMEM((8, 128), jnp.float32),
        pltpu.VMEM((8, 128), jnp.float32),
        pltpu.VMEM((8, 128), jnp.float32),
        pltpu.VMEM((8, 128), jnp.float32),
        pltpu.VMEM((16,), jnp.float32),
        pltpu.SemaphoreType.DMA,
        pltpu.SemaphoreType.DMA,
        pltpu.SemaphoreType.DMA,
        pltpu.SemaphoreType.DMA,
        pltpu.SemaphoreType.DMA,
        pltpu.SemaphoreType.DMA,
        pltpu.SemaphoreType.DMA,
    ],
)(_sc_body)


def kernel(idx, targets, table):
    idx_f = jnp.pad(idx.reshape(-1).astype(jnp.int32), (0, IDX_PAD - ROWS_N))
    tgt_f = jnp.pad(targets.reshape(-1).astype(jnp.int32),
                    (0, IDX_PAD - ROWS_N))
    cpad = VPAD - VOCAB_N
    tpad_sq = jnp.pad(table, ((0, cpad), (0, cpad)), constant_values=-1e30)
    tflat = jnp.pad(table, ((0, 0), (0, cpad))).reshape(-1)
    gflat = jnp.pad(table.T, ((0, 0), (0, cpad))).reshape(-1)
    lse = pl.pallas_call(
        _lse_body,
        out_shape=jax.ShapeDtypeStruct((VPAD,), jnp.float32),
    )(tpad_sq)
    logits_t, part = _sc_gather(idx_f, tgt_f, lse, tflat, gflat)
    loss = pl.pallas_call(
        _mean_body,
        out_shape=jax.ShapeDtypeStruct((1, 1), jnp.float32),
    )(part)
    return logits_t.T, loss[0, 0]


# speedup vs baseline: 4.5436x; 1.1157x over previous
import functools

import jax
import jax.numpy as jnp
from jax import lax
from jax.experimental import pallas as pl
from jax.experimental.pallas import tpu as pltpu
from jax.experimental.pallas import tpu_sc as plsc

VOCAB_N = 1000
VPAD = 1024
ROWS_N = 51200
NC = 2
NS = 16
NW = NC * NS
PER_W = ROWS_N // NW
LCH = 80
NVG = VOCAB_N // 8
NU = ROWS_N // 128
IDX_PAD = 51456


def _lse_body(tpad_ref, out_ref):
    x = tpad_ref[...]
    m = jnp.max(x, axis=1)
    s = jnp.sum(jnp.exp(x - m[:, None]), axis=1)
    out_ref[...] = m + jnp.log(s)


def _mean_body(part_ref, out_ref):
    out_ref[...] = (jnp.sum(part_ref[...]) * (1.0 / ROWS_N))[None, None]


def _sc_body(idx_hbm, tgt_hbm, lse_hbm, tflat_hbm, gflat_hbm,
             out_hbm, part_hbm,
             idxg_v, idx_v, tgt_v, fidx_v, lsei_v, tlog_v,
             g0, g1, st0, st1, st2, st3, part_v, gsh,
             semg0, semg1, semw0, semw1, semw2, semw3, seml):
    wid = lax.axis_index("s") * NC + lax.axis_index("c")

    @pl.when(lax.axis_index("s") == 0)
    def _():
        pltpu.sync_copy(gflat_hbm, gsh)

    plsc.subcore_barrier()

    base = wid * PER_W
    pltpu.sync_copy(idx_hbm.at[pl.ds(base, PER_W)], idx_v)
    pltpu.sync_copy(tgt_hbm.at[pl.ds(base, PER_W)], tgt_v)

    def fidx_body(j, _):
        s = pl.ds(j * 16, 16)
        fidx_v[s] = idx_v[s] * VPAD + tgt_v[s]
        return 0

    lax.fori_loop(0, PER_W // 16, fidx_body, 0)

    def lgather_body(j, _):
        s = pl.ds(j * LCH, LCH)
        pltpu.async_copy(lse_hbm.at[idx_v.at[s]], lsei_v.at[s], seml).wait()
        pltpu.async_copy(tflat_hbm.at[fidx_v.at[s]], tlog_v.at[s],
                         seml).wait()
        return 0

    lax.fori_loop(0, PER_W // LCH, lgather_body, 0)

    def loss_body(j, acc):
        s = pl.ds(j * 16, 16)
        return acc + (lsei_v[s] - tlog_v[s])

    acc = lax.fori_loop(0, PER_W // 16, loss_body,
                        jnp.zeros((16,), jnp.float32))
    part_v[...] = acc
    pltpu.sync_copy(part_v, part_hbm.at[wid])

    nu = jnp.where(wid < 16, 13, 12)
    u0 = jnp.where(wid < 16, wid * 13, wid * 12 + 16)
    i0 = u0 * 128
    pltpu.sync_copy(idx_hbm.at[pl.ds(i0, 1664)], idxg_v)

    gbuf = (g0, g1)
    semg = (semg0, semg1)
    stage = (st0, st1, st2, st3)
    semw = (semw0, semw1, semw2, semw3)

    def g_load(vg, k):
        return pltpu.async_copy(
            gsh.at[pl.ds(vg * (8 * VPAD), 8 * VPAD)], gbuf[k], semg[k])

    g_load(0, 0)

    def build_tile(gk, vg, t, s):
        gcount = vg * nu + t

        @pl.when(gcount >= 4)
        def _():
            pltpu.make_async_copy(
                stage[s],
                out_hbm.at[pl.ds(pl.multiple_of(vg * 8, 8), 8),
                           pl.ds(pl.multiple_of(i0 + t * 128, 128), 128)],
                semw[s]).wait()

        @plsc.parallel_loop(0, 8, step=1, unroll=8)
        def _(grp):
            iv = idxg_v[pl.ds(t * 128 + grp * 16, 16)]
            for vr in range(8):
                stage[s][vr, pl.ds(grp * 16, 16)] = plsc.load_gather(
                    gbuf[gk], [iv + vr * VPAD])
        pltpu.async_copy(
            stage[s],
            out_hbm.at[pl.ds(pl.multiple_of(vg * 8, 8), 8),
                       pl.ds(pl.multiple_of(i0 + t * 128, 128), 128)],
            semw[s])

    def do_vgroup(gk, vg):
        @pl.when(vg + 1 < NVG)
        def _():
            g_load(vg + 1, 1 - gk)

        pltpu.make_async_copy(
            gsh.at[pl.ds(vg * (8 * VPAD), 8 * VPAD)],
            gbuf[gk], semg[gk]).wait()

        def unit4(j, _):
            for u in range(4):
                build_tile(gk, vg, j * 4 + u, u)
            return 0

        lax.fori_loop(0, 3, unit4, 0)

        @pl.when(nu == 13)
        def _():
            build_tile(gk, vg, 12, 0)

    def vg_pair(o, _):
        @pl.when(o * 2 < NVG)
        def _():
            do_vgroup(0, o * 2)

        @pl.when(o * 2 + 1 < NVG)
        def _():
            do_vgroup(1, o * 2 + 1)

        return 0

    lax.fori_loop(0, (NVG + 1) // 2, vg_pair, 0)

    for s in range(4):
        pltpu.make_async_copy(
            stage[s],
            out_hbm.at[pl.ds(pl.multiple_of(0 * 8, 8), 8),
                       pl.ds(pl.multiple_of(i0, 128), 128)],
            semw[s]).wait()


_sc_gather = functools.partial(
    pl.kernel,
    out_type=[
        jax.ShapeDtypeStruct((VOCAB_N, ROWS_N), jnp.float32),
        jax.ShapeDtypeStruct((NW, 16), jnp.float32),
    ],
    mesh=plsc.VectorSubcoreMesh(core_axis_name="c", subcore_axis_name="s"),
    compiler_params=pltpu.CompilerParams(use_tc_tiling_on_sc=True,
                                         needs_layout_passes=False,
                                         disable_bounds_checks=True),
    scratch_types=[
        pltpu.VMEM((1664,), jnp.int32),
        pltpu.VMEM((PER_W,), jnp.int32),
        pltpu.VMEM((PER_W,), jnp.int32),
        pltpu.VMEM((PER_W,), jnp.int32),
        pltpu.VMEM((PER_W,), jnp.float32),
        pltpu.VMEM((PER_W,), jnp.float32),
        pltpu.VMEM((8 * VPAD,), jnp.float32),
        pltpu.VMEM((8 * VPAD,), jnp.float32),
        pltpu.VMEM((8, 128), jnp.float32),
        pltpu.VMEM((8, 128), jnp.float32),
        pltpu.VMEM((8, 128), jnp.float32),
        pltpu.VMEM((8, 128), jnp.float32),
        pltpu.VMEM((16,), jnp.float32),
        pltpu.VMEM_SHARED((NVG * 8 * VPAD,), jnp.float32),
        pltpu.SemaphoreType.DMA,
        pltpu.SemaphoreType.DMA,
        pltpu.SemaphoreType.DMA,
        pltpu.SemaphoreType.DMA,
        pltpu.SemaphoreType.DMA,
        pltpu.SemaphoreType.DMA,
        pltpu.SemaphoreType.DMA,
    ],
)(_sc_body)


def kernel(idx, targets, table):
    idx_f = jnp.pad(idx.reshape(-1).astype(jnp.int32), (0, IDX_PAD - ROWS_N))
    tgt_f = jnp.pad(targets.reshape(-1).astype(jnp.int32),
                    (0, IDX_PAD - ROWS_N))
    cpad = VPAD - VOCAB_N
    tpad_sq = jnp.pad(table, ((0, cpad), (0, cpad)), constant_values=-1e30)
    tflat = jnp.pad(table, ((0, 0), (0, cpad))).reshape(-1)
    gflat = jnp.pad(table.T, ((0, 0), (0, cpad))).reshape(-1)
    lse = pl.pallas_call(
        _lse_body,
        out_shape=jax.ShapeDtypeStruct((VPAD,), jnp.float32),
    )(tpad_sq)
    logits_t, part = _sc_gather(idx_f, tgt_f, lse, tflat, gflat)
    loss = pl.pallas_call(
        _mean_body,
        out_shape=jax.ShapeDtypeStruct((1, 1), jnp.float32),
    )(part)
    return logits_t.T, loss[0, 0]


# speedup vs baseline: 4.7614x; 1.0479x over previous
import functools

import jax
import jax.numpy as jnp
from jax import lax
from jax.experimental import pallas as pl
from jax.experimental.pallas import tpu as pltpu
from jax.experimental.pallas import tpu_sc as plsc

VOCAB_N = 1000
VPAD = 1024
ROWS_N = 51200
NC = 2
NS = 16
NW = NC * NS
PER_W = ROWS_N // NW
LCH = 80
NVG = VOCAB_N // 8
NU = ROWS_N // 128
IDX_PAD = 51456


def _lse_body(tpad_ref, out_ref):
    x = tpad_ref[...]
    m = jnp.max(x, axis=1)
    s = jnp.sum(jnp.exp(x - m[:, None]), axis=1)
    out_ref[...] = m + jnp.log(s)


def _mean_body(part_ref, out_ref):
    out_ref[...] = (jnp.sum(part_ref[...]) * (1.0 / ROWS_N))[None, None]


def _sc_body(idx_hbm, tgt_hbm, lse_hbm, tflat_hbm, gflat_hbm,
             out_hbm, part_hbm,
             idxg_v, idx_v, tgt_v, fidx_v, lsei_v, tlog_v,
             g0, g1, st0, st1, st2, st3, part_v, gsh,
             semg0, semg1, semw0, semw1, semw2, semw3, seml):
    wid = lax.axis_index("s") * NC + lax.axis_index("c")

    @pl.when(lax.axis_index("s") == 0)
    def _():
        pltpu.sync_copy(gflat_hbm, gsh)

    plsc.subcore_barrier()

    base = wid * PER_W
    pltpu.sync_copy(idx_hbm.at[pl.ds(base, PER_W)], idx_v)
    pltpu.sync_copy(tgt_hbm.at[pl.ds(base, PER_W)], tgt_v)

    def fidx_body(j, _):
        s = pl.ds(j * 16, 16)
        fidx_v[s] = idx_v[s] * VPAD + tgt_v[s]
        return 0

    lax.fori_loop(0, PER_W // 16, fidx_body, 0)

    def lgather_body(j, _):
        s = pl.ds(j * LCH, LCH)
        pltpu.async_copy(lse_hbm.at[idx_v.at[s]], lsei_v.at[s], seml)
        pltpu.async_copy(tflat_hbm.at[fidx_v.at[s]], tlog_v.at[s], seml)
        return 0

    lax.fori_loop(0, PER_W // LCH, lgather_body, 0)

    nu = jnp.where(wid < 16, 13, 12)
    u0 = jnp.where(wid < 16, wid * 13, wid * 12 + 16)
    i0 = u0 * 128
    pltpu.sync_copy(idx_hbm.at[pl.ds(i0, 1664)], idxg_v)

    gbuf = (g0, g1)
    semg = (semg0, semg1)
    stage = (st0, st1, st2, st3)
    semw = (semw0, semw1, semw2, semw3)

    def g_load(vg, k):
        return pltpu.async_copy(
            gsh.at[pl.ds(vg * (8 * VPAD), 8 * VPAD)], gbuf[k], semg[k])

    g_load(0, 0)

    def build_tile(gk, vg, t, s):
        gcount = vg * nu + t

        @pl.when(gcount >= 4)
        def _():
            pltpu.make_async_copy(
                stage[s],
                out_hbm.at[pl.ds(pl.multiple_of(vg * 8, 8), 8),
                           pl.ds(pl.multiple_of(i0 + t * 128, 128), 128)],
                semw[s]).wait()

        @plsc.parallel_loop(0, 8, step=1, unroll=8)
        def _(grp):
            iv = idxg_v[pl.ds(t * 128 + grp * 16, 16)]
            for vr in range(8):
                stage[s][vr, pl.ds(grp * 16, 16)] = plsc.load_gather(
                    gbuf[gk], [iv + vr * VPAD])
        pltpu.async_copy(
            stage[s],
            out_hbm.at[pl.ds(pl.multiple_of(vg * 8, 8), 8),
                       pl.ds(pl.multiple_of(i0 + t * 128, 128), 128)],
            semw[s])

    def do_vgroup(gk, vg):
        @pl.when(vg + 1 < NVG)
        def _():
            g_load(vg + 1, 1 - gk)

        pltpu.make_async_copy(
            gsh.at[pl.ds(vg * (8 * VPAD), 8 * VPAD)],
            gbuf[gk], semg[gk]).wait()

        def unit4(j, _):
            for u in range(4):
                build_tile(gk, vg, j * 4 + u, u)
            return 0

        lax.fori_loop(0, 3, unit4, 0)

        @pl.when(nu == 13)
        def _():
            build_tile(gk, vg, 12, 0)

    def vg_pair(o, _):
        @pl.when(o * 2 < NVG)
        def _():
            do_vgroup(0, o * 2)

        @pl.when(o * 2 + 1 < NVG)
        def _():
            do_vgroup(1, o * 2 + 1)

        return 0

    lax.fori_loop(0, (NVG + 1) // 2, vg_pair, 0)

    for s in range(4):
        pltpu.make_async_copy(
            stage[s],
            out_hbm.at[pl.ds(pl.multiple_of(0 * 8, 8), 8),
                       pl.ds(pl.multiple_of(i0, 128), 128)],
            semw[s]).wait()

    def ldrain_body(j, _):
        s = pl.ds(j * LCH, LCH)
        pltpu.make_async_copy(
            lse_hbm.at[idx_v.at[s]], lsei_v.at[s], seml).wait()
        pltpu.make_async_copy(
            tflat_hbm.at[fidx_v.at[s]], tlog_v.at[s], seml).wait()
        return 0

    lax.fori_loop(0, PER_W // LCH, ldrain_body, 0)

    def loss_body(j, acc):
        s = pl.ds(j * 16, 16)
        return acc + (lsei_v[s] - tlog_v[s])

    acc = lax.fori_loop(0, PER_W // 16, loss_body,
                        jnp.zeros((16,), jnp.float32))
    part_v[...] = acc
    pltpu.sync_copy(part_v, part_hbm.at[wid])


_sc_gather = functools.partial(
    pl.kernel,
    out_type=[
        jax.ShapeDtypeStruct((VOCAB_N, ROWS_N), jnp.float32),
        jax.ShapeDtypeStruct((NW, 16), jnp.float32),
    ],
    mesh=plsc.VectorSubcoreMesh(core_axis_name="c", subcore_axis_name="s"),
    compiler_params=pltpu.CompilerParams(use_tc_tiling_on_sc=True,
                                         needs_layout_passes=False,
                                         disable_bounds_checks=True),
    scratch_types=[
        pltpu.VMEM((1664,), jnp.int32),
        pltpu.VMEM((PER_W,), jnp.int32),
        pltpu.VMEM((PER_W,), jnp.int32),
        pltpu.VMEM((PER_W,), jnp.int32),
        pltpu.VMEM((PER_W,), jnp.float32),
        pltpu.VMEM((PER_W,), jnp.float32),
        pltpu.VMEM((8 * VPAD,), jnp.float32),
        pltpu.VMEM((8 * VPAD,), jnp.float32),
        pltpu.VMEM((8, 128), jnp.float32),
        pltpu.VMEM((8, 128), jnp.float32),
        pltpu.VMEM((8, 128), jnp.float32),
        pltpu.VMEM((8, 128), jnp.float32),
        pltpu.VMEM((16,), jnp.float32),
        pltpu.VMEM_SHARED((NVG * 8 * VPAD,), jnp.float32),
        pltpu.SemaphoreType.DMA,
        pltpu.SemaphoreType.DMA,
        pltpu.SemaphoreType.DMA,
        pltpu.SemaphoreType.DMA,
        pltpu.SemaphoreType.DMA,
        pltpu.SemaphoreType.DMA,
        pltpu.SemaphoreType.DMA,
    ],
)(_sc_body)


def kernel(idx, targets, table):
    idx_f = jnp.pad(idx.reshape(-1).astype(jnp.int32), (0, IDX_PAD - ROWS_N))
    tgt_f = jnp.pad(targets.reshape(-1).astype(jnp.int32),
                    (0, IDX_PAD - ROWS_N))
    cpad = VPAD - VOCAB_N
    tpad_sq = jnp.pad(table, ((0, cpad), (0, cpad)), constant_values=-1e30)
    tflat = jnp.pad(table, ((0, 0), (0, cpad))).reshape(-1)
    gflat = jnp.pad(table.T, ((0, 0), (0, cpad))).reshape(-1)
    lse = pl.pallas_call(
        _lse_body,
        out_shape=jax.ShapeDtypeStruct((VPAD,), jnp.float32),
    )(tpad_sq)
    logits_t, part = _sc_gather(idx_f, tgt_f, lse, tflat, gflat)
    loss = pl.pallas_call(
        _mean_body,
        out_shape=jax.ShapeDtypeStruct((1, 1), jnp.float32),
    )(part)
    return logits_t.T, loss[0, 0]
